# baseline (speedup 1.0000x reference)
import jax
import jax.numpy as jnp
from jax.experimental import pallas as pl

N = 10000
H = 2


def _gat(x, edge_index, W, a_src, a_dst, bias, num_nodes, heads, out_ch):
    loop = jnp.arange(num_nodes, dtype=edge_index.dtype)
    ei = jnp.concatenate([edge_index, jnp.stack([loop, loop])], axis=1)
    src, dst = ei[0], ei[1]
    h = (x @ W).reshape(num_nodes, heads, out_ch)
    alpha_src = (h * a_src).sum(-1)
    alpha_dst = (h * a_dst).sum(-1)
    alpha = alpha_src[src] + alpha_dst[dst]
    alpha = jax.nn.leaky_relu(alpha, negative_slope=0.2)
    amax = jax.ops.segment_max(alpha, dst, num_segments=num_nodes)
    amax = jnp.where(jnp.isfinite(amax), amax, 0.0)
    alpha = jnp.exp(alpha - amax[dst])
    denom = jax.ops.segment_sum(alpha, dst, num_segments=num_nodes)
    alpha = alpha / (denom[dst] + 1e-16)
    msg = h[src] * alpha[:, :, None]
    out = jax.ops.segment_sum(msg, dst, num_segments=num_nodes)
    return out.reshape(num_nodes, heads * out_ch) + bias


def _addb(x_ref, b_ref, o_ref):
    o_ref[...] = x_ref[...] + b_ref[...]


def kernel(x, undirected_edges_small, directed_edges_small,
           W1, a_src1, a_dst1, b1, W2, a_src2, a_dst2, b2):
    h1 = _gat(x, undirected_edges_small, W1, a_src1, a_dst1, b1, N, H, 128)
    h2 = _gat(h1, directed_edges_small, W2, a_src2, a_dst2, jnp.zeros_like(b2), N, H, 256)
    out = pl.pallas_call(
        _addb,
        out_shape=jax.ShapeDtypeStruct(h2.shape, h2.dtype),
    )(h2, jnp.broadcast_to(b2, h2.shape))
    return out


# speedup vs baseline: 32.7054x; 32.7054x over previous
import functools

import jax
import jax.numpy as jnp
from jax import lax
from jax.experimental import pallas as pl
from jax.experimental.pallas import tpu as pltpu
from jax.experimental.pallas import tpu_sc as plsc

N = 10000
E = 320000
D = 128
H = 2
ETRUE = E + N
EPAD = 331776
NC, NS, L = 2, 16, 16
CH1 = EPAD // (NC * NS)
CH2 = EPAD // NS
NB = CH2 // 128
NPAD = 10240
RPT = NPAD // NS
RCH = 128


def _prep_body(x_ref, bin_ref, w_ref, as_ref, ad_ref, h_ref, ps_ref, pd_ref):
    sin = x_ref.shape[0]
    if sin == 1:
        xb = x_ref[0]
    else:
        xb = jnp.concatenate([x_ref[i] for i in range(sin)], axis=1)
    xb = xb + bin_ref[...]
    hb = jnp.dot(xb, w_ref[...], preferred_element_type=jnp.float32,
                 precision=lax.Precision.HIGHEST)
    h_ref[...] = hb
    ps_ref[0, 0, :] = jnp.sum(hb * as_ref[0], axis=1)
    pd_ref[0, 0, :] = jnp.sum(hb * ad_ref[0], axis=1)


def _make_prep(sin, s):
    din = sin * 128
    return pl.pallas_call(
        _prep_body,
        grid=(s,),
        in_specs=[
            pl.BlockSpec((sin, N, 128), lambda i: (0, 0, 0)),
            pl.BlockSpec((1, din), lambda i: (0, 0)),
            pl.BlockSpec((din, 128), lambda i: (0, i)),
            pl.BlockSpec((1, 1, 128), lambda i: (i, 0, 0)),
            pl.BlockSpec((1, 1, 128), lambda i: (i, 0, 0)),
        ],
        out_specs=[
            pl.BlockSpec((N, 128), lambda i: (i, 0)),
            pl.BlockSpec((1, 1, N), lambda i: (i, 0, 0)),
            pl.BlockSpec((1, 1, N), lambda i: (i, 0, 0)),
        ],
        out_shape=[
            jax.ShapeDtypeStruct((s * N, 128), jnp.float32),
            jax.ShapeDtypeStruct((s, 1, N), jnp.float32),
            jax.ShapeDtypeStruct((s, 1, N), jnp.float32),
        ],
    )


def _asm_body(x_ref, b_ref, o_ref):
    o_ref[...] = x_ref[0] + b_ref[0]


def _make_asm(s):
    return pl.pallas_call(
        _asm_body,
        grid=(s,),
        in_specs=[
            pl.BlockSpec((1, N, 128), lambda i: (i, 0, 0)),
            pl.BlockSpec((1, 1, 128), lambda i: (i, 0, 0)),
        ],
        out_specs=pl.BlockSpec((N, 128), lambda i: (0, i)),
        out_shape=jax.ShapeDtypeStruct((N, s * 128), jnp.float32),
    )


def _vec_max(tbl):
    def body(i, acc):
        return jnp.maximum(acc, tbl[pl.ds(i * L, L)])
    v = lax.fori_loop(0, N // L, body, jnp.full((L,), -3.0e38, jnp.float32))
    m = v[0]
    for jj in range(1, L):
        m = jnp.maximum(m, v[jj])
    return m


def _make_ph1(s):
    hs = s // 2
    mesh = plsc.VectorSubcoreMesh(core_axis_name="c", subcore_axis_name="s")

    @functools.partial(
        pl.kernel,
        out_type=jax.ShapeDtypeStruct((2 * EPAD,), jnp.float32),
        mesh=mesh,
        compiler_params=pltpu.CompilerParams(needs_layout_passes=False),
        scratch_types=[
            pltpu.VMEM((N,), jnp.float32),
            pltpu.VMEM((N,), jnp.float32),
            pltpu.VMEM((N,), jnp.float32),
            pltpu.VMEM((N,), jnp.float32),
            pltpu.VMEM((N,), jnp.float32),
            pltpu.VMEM((CH1,), jnp.int32),
            pltpu.VMEM((CH1,), jnp.int32),
            pltpu.VMEM((CH1,), jnp.float32),
            pltpu.VMEM((CH1,), jnp.float32),
        ],
    )
    def ph1(srcp, dstp, psrc, pdst, expo,
            as0, as1, ad0, ad1, tmp, srcb, dstb, e0, e1):
        cid = lax.axis_index("c")
        sid = lax.axis_index("s")
        wid = sid * NC + cid
        base = wid * CH1

        for tbl, p, h in ((as0, psrc, 0), (as1, psrc, 1),
                          (ad0, pdst, 0), (ad1, pdst, 1)):
            pltpu.sync_copy(p.at[pl.ds(h * hs * N, N)], tbl)
            for j in range(1, hs):
                pltpu.sync_copy(p.at[pl.ds((h * hs + j) * N, N)], tmp)
                def addl(i, _, tbl=tbl):
                    tbl[pl.ds(i * L, L)] = tbl[pl.ds(i * L, L)] + tmp[pl.ds(i * L, L)]
                    return 0
                lax.fori_loop(0, N // L, addl, 0)

        m0 = _vec_max(as0) + _vec_max(ad0)
        m0 = jnp.where(m0 > 0, m0, 0.2 * m0)
        m1 = _vec_max(as1) + _vec_max(ad1)
        m1 = jnp.where(m1 > 0, m1, 0.2 * m1)

        pltpu.sync_copy(srcp.at[pl.ds(base, CH1)], srcb)
        pltpu.sync_copy(dstp.at[pl.ds(base, CH1)], dstb)
        iota = lax.iota(jnp.int32, L)

        def grp(i, _):
            s16 = srcb[pl.ds(i * L, L)]
            d16 = dstb[pl.ds(i * L, L)]
            valid = (base + i * L + iota) < ETRUE
            for ast, adt, eb, m in ((as0, ad0, e0, m0), (as1, ad1, e1, m1)):
                a = plsc.load_gather(ast, [s16]) + plsc.load_gather(adt, [d16])
                a = jnp.where(a > 0, a, 0.2 * a)
                e = jnp.exp(a - m)
                eb[pl.ds(i * L, L)] = jnp.where(valid, e, 0.0)
            return 0

        lax.fori_loop(0, CH1 // L, grp, 0)
        pltpu.sync_copy(e0, expo.at[pl.ds(base, CH1)])
        pltpu.sync_copy(e1, expo.at[pl.ds(EPAD + base, CH1)])

    return ph1


G = 6
NBG = NB // G


def _make_ph2(s):
    hs = s // 2
    mesh = plsc.VectorSubcoreMesh(core_axis_name="c", subcore_axis_name="s")

    @functools.partial(
        pl.kernel,
        out_type=jax.ShapeDtypeStruct((s, NPAD, 128), jnp.float32),
        mesh=mesh,
        compiler_params=pltpu.CompilerParams(needs_layout_passes=False),
        scratch_types=[
            pltpu.VMEM_SHARED((NPAD, 128), jnp.float32),
            pltpu.VMEM((G * 128,), jnp.int32),
            pltpu.VMEM((G, 128), jnp.int32),
            pltpu.VMEM((G, 128), jnp.int32),
            pltpu.VMEM((G * 128,), jnp.float32),
            pltpu.VMEM((128, 128), jnp.float32),
            pltpu.VMEM((RPT,), jnp.float32),
            pltpu.SemaphoreType.DMA,
        ],
    )
    def ph2(srcp, dstp, expv, hlin, out,
            acc, stage, sgid, sdst, sexp, rbuf, dnp, sem):
        cid = lax.axis_index("c")
        sid = lax.axis_index("s")
        rbase = sid * RPT
        ebase = sid * CH2
        zero = jnp.zeros((L,), jnp.float32)
        iota = lax.iota(jnp.int32, L)
        zidx = jnp.zeros((L,), jnp.int32)

        def zrbuf():
            def z(i, _):
                rbuf[i // 8, pl.ds((i % 8) * L, L)] = zero
                return 0
            lax.fori_loop(0, RCH * 8, z, 0)

        def zacc():
            zrbuf()
            for k in range(5):
                pltpu.sync_copy(rbuf, acc.at[pl.ds(rbase + RCH * k, RCH)])

        def load_dst(gb):
            pltpu.sync_copy(dstp.at[pl.ds(ebase + gb * (G * 128), G * 128)],
                            stage)

            def mkd(i, _):
                sdst[i // 8, pl.ds((i % 8) * L, L)] = stage[pl.ds(i * L, L)]
                return 0
            lax.fori_loop(0, G * 8, mkd, 0)

        def load_exp(gb):
            pltpu.sync_copy(
                expv.at[pl.ds(cid * EPAD + ebase + gb * (G * 128), G * 128)],
                sexp)

        zacc()
        plsc.subcore_barrier()

        def dgroup(gb, _):
            load_dst(gb)
            load_exp(gb)

            def batch(j, _):
                def fill(g, _):
                    w16 = sexp[pl.ds(j * 128 + g * L, L)]
                    for jj in range(L):
                        w = w16[jj]
                        for u in range(8):
                            rbuf[g * L + jj, pl.ds(u * L, L)] = zero + w
                    return 0
                lax.fori_loop(0, 8, fill, 0)
                pltpu.sync_copy(rbuf, acc.at[sdst.at[j]], add=True)
                return 0

            lax.fori_loop(0, G, batch, 0)
            return 0

        lax.fori_loop(0, NBG, dgroup, 0)
        plsc.subcore_barrier()

        for k in range(5):
            pltpu.sync_copy(acc.at[pl.ds(rbase + RCH * k, RCH)], rbuf)

            def dext(g, _, k=k):
                d16 = plsc.load_gather(rbuf, [g * L + iota, zidx])
                dnp[pl.ds(k * RCH + g * L, L)] = d16
                return 0
            lax.fori_loop(0, 8, dext, 0)

        zacc()
        plsc.subcore_barrier()

        for sl in range(hs):
            bias = (cid * hs + sl) * N

            def group(gb, _, bias=bias):
                pltpu.sync_copy(srcp.at[pl.ds(ebase + gb * (G * 128), G * 128)],
                                stage)

                def mkg(i, _):
                    sgid[i // 8, pl.ds((i % 8) * L, L)] = (
                        stage[pl.ds(i * L, L)] + bias)
                    return 0
                lax.fori_loop(0, G * 8, mkg, 0)

                load_dst(gb)
                load_exp(gb)

                def batch(j, _):
                    pltpu.async_copy(hlin.at[sgid.at[j]], rbuf, sem).wait()

                    def scale(g, _):
                        w16 = sexp[pl.ds(j * 128 + g * L, L)]
                        for jj in range(L):
                            w = w16[jj]
                            for u in range(8):
                                rbuf[g * L + jj, pl.ds(u * L, L)] = (
                                    rbuf[g * L + jj, pl.ds(u * L, L)] * w)
                        return 0

                    lax.fori_loop(0, 8, scale, 0)
                    pltpu.sync_copy(rbuf, acc.at[sdst.at[j]], add=True)
                    return 0

                lax.fori_loop(0, G, batch, 0)
                return 0

            lax.fori_loop(0, NBG, group, 0)
            plsc.subcore_barrier()

            for k in range(5):
                r0 = rbase + RCH * k
                pltpu.sync_copy(acc.at[pl.ds(r0, RCH)], rbuf)

                def norm(g, _, k=k):
                    d16 = dnp[pl.ds(k * RCH + g * L, L)]
                    winv = 1.0 / (d16 + 1e-30)
                    for jj in range(L):
                        w = winv[jj]
                        for u in range(8):
                            rbuf[g * L + jj, pl.ds(u * L, L)] = (
                                rbuf[g * L + jj, pl.ds(u * L, L)] * w)
                    return 0

                lax.fori_loop(0, 8, norm, 0)
                pltpu.sync_copy(rbuf, out.at[cid * hs + sl, pl.ds(r0, RCH)])

            if sl + 1 < hs:
                zacc()
                plsc.subcore_barrier()

    return ph2


_PREP1 = _make_prep(1, 2)
_PREP2 = _make_prep(2, 4)
_ASM = _make_asm(4)
_PH1_A = _make_ph1(2)
_PH1_B = _make_ph1(4)
_PH2_A = _make_ph2(2)
_PH2_B = _make_ph2(4)


def _pad_edges(edge_index):
    loop = jnp.arange(N, dtype=jnp.int32)
    pad = jnp.zeros((EPAD - ETRUE,), jnp.int32)
    src = jnp.concatenate([edge_index[0].astype(jnp.int32), loop, pad])
    dst = jnp.concatenate([edge_index[1].astype(jnp.int32), loop, pad])
    return src, dst


def _gat_layer(x_parts, bias_in, edge_index, W, a_src, a_dst, prep, ph1, ph2, s):
    src, dst = _pad_edges(edge_index)
    hlin, psrc, pdst = prep(x_parts, bias_in.reshape(1, -1), W,
                            a_src.reshape(s, 1, 128), a_dst.reshape(s, 1, 128))
    expv = ph1(src, dst, psrc.reshape(s * N), pdst.reshape(s * N))
    out = ph2(src, dst, expv, hlin)
    return out[:, :N, :]


def kernel(x, undirected_edges_small, directed_edges_small,
           W1, a_src1, a_dst1, b1, W2, a_src2, a_dst2, b2):
    zero_in = jnp.zeros((128,), jnp.float32)
    out1 = _gat_layer(x.reshape(1, N, 128), zero_in, undirected_edges_small,
                      W1, a_src1, a_dst1, _PREP1, _PH1_A, _PH2_A, 2)
    out2 = _gat_layer(out1, b1, directed_edges_small,
                      W2, a_src2, a_dst2, _PREP2, _PH1_B, _PH2_B, 4)
    return _ASM(out2, b2.reshape(4, 1, 128))
